# baseline (speedup 1.0000x reference)
import functools

import jax
import jax.numpy as jnp
from jax import lax
from jax.experimental import pallas as pl
from jax.experimental.pallas import tpu as pltpu
from jax.experimental.pallas import tpu_sc as plsc

NC = 2
NS = 16
NW = NC * NS
CH = 128
D = 32
NBUF = 6


PB = 6


@jax.jit
def _sc_format(Wt):
    V = Wt.shape[1]
    n_full = V // 128
    rem = V - n_full * 128
    n_groups = (n_full // NW + PB) // PB
    mesh = plsc.VectorSubcoreMesh(core_axis_name="c", subcore_axis_name="s")

    @functools.partial(
        pl.kernel,
        out_type=jax.ShapeDtypeStruct((V * D // 128, 128), jnp.float32),
        mesh=mesh,
        scratch_types=[
            pltpu.VMEM((PB, D, 128), jnp.float32),
            pltpu.VMEM((PB, D, 128), jnp.float32),
            pltpu.VMEM((D, rem), jnp.float32),
            pltpu.VMEM((rem * D // 128, 128), jnp.float32),
            pltpu.SemaphoreType.DMA((PB,)),
            pltpu.SemaphoreType.DMA((PB,)),
        ],
        compiler_params=pltpu.CompilerParams(
            use_tc_tiling_on_sc=True, needs_layout_passes=False),
    )
    def fmt_kernel(wt_hbm, out_hbm, w32, t128, w64, t64, rsem, wsem):
        wid = lax.axis_index("s") * NC + lax.axis_index("c")
        lane = jnp.arange(16, dtype=jnp.int32)

        def r_copy(u, b):
            return pltpu.make_async_copy(
                wt_hbm.at[:, pl.ds(u * 128, 128)], w32.at[b], rsem.at[b])

        def w_copy(u, b):
            return pltpu.make_async_copy(
                t128.at[b], out_hbm.at[pl.ds(u * D, D)], wsem.at[b])

        def transpose(src, dst, ni):
            @plsc.parallel_loop(0, 16, unroll=4)
            def _(s):
                shift = (lane + s) & 15
                lo = lane * D + shift
                for db in range(D // 16):
                    col_d = 16 * db + shift
                    hi0 = (lo + 16 * db) >> 7
                    lo0 = (lo + 16 * db) & 127
                    for ib in range(ni // 16):
                        row_i = lane + 16 * ib
                        vec = plsc.load_gather(src, [col_d, row_i])
                        plsc.store_scatter(
                            dst, [hi0 + (16 * D // 128) * ib, lo0], vec)

        for b in range(PB):
            @pl.when(wid + NW * b < n_full)
            def _():
                r_copy(wid + NW * b, b).start()

        @pl.loop(0, n_groups)
        def _(g):
            for b in range(PB):
                u = wid + NW * (g * PB + b)

                @pl.when(u < n_full)
                def _():
                    r_copy(u, b).wait()

                    @pl.when(u >= NW * PB)
                    def _():
                        w_copy(u - NW * PB, b).wait()

                    transpose(w32.at[b], t128.at[b], 128)
                    w_copy(u, b).start()

                    @pl.when(u + NW * PB < n_full)
                    def _():
                        r_copy(u + NW * PB, b).start()

        for b in range(PB):
            w_copy(0, b).wait()

        if rem:
            @pl.when(wid == lax.rem(n_full, NW))
            def _():
                pltpu.sync_copy(wt_hbm.at[:, pl.ds(n_full * 128, rem)], w64)
                transpose(w64, t64, rem)
                pltpu.sync_copy(
                    t64, out_hbm.at[pl.ds(n_full * D, rem * D // 128)])

    return fmt_kernel(Wt)


@jax.jit
def _sc_gather_t(W_embed, idx):
    n_t, n_c = 200, 32
    n_units = n_t * n_c
    du = n_units // NW
    n_groups = du // NBUF
    mesh = plsc.VectorSubcoreMesh(core_axis_name="c", subcore_axis_name="s")

    @functools.partial(
        pl.kernel,
        out_type=jax.ShapeDtypeStruct((n_t, D // 8, n_c, 8, CH), jnp.float32),
        mesh=mesh,
        scratch_types=[
            pltpu.VMEM((du, CH), jnp.int32),
            pltpu.VMEM((NBUF, CH, D), jnp.float32),
            pltpu.VMEM((NBUF, D, CH), jnp.float32),
            pltpu.SemaphoreType.DMA((NBUF,)),
            pltpu.SemaphoreType.DMA((NBUF,)),
        ],
        compiler_params=pltpu.CompilerParams(
            use_tc_tiling_on_sc=False, needs_layout_passes=False),
    )
    def gather_kernel(table_hbm, idx_hbm, out_hbm, idx_v, rows_v, tv_v, gsem, ssem):
        wid = lax.axis_index("s") * NC + lax.axis_index("c")
        u0 = wid * du
        pltpu.sync_copy(idx_hbm.at[wid], idx_v)

        lane = jnp.arange(16, dtype=jnp.int32)

        def g_copy(j, b):
            return pltpu.make_async_copy(
                table_hbm.at[idx_v.at[j]], rows_v.at[b], gsem.at[b])

        def s_copies(j, b):
            u = u0 + j
            t = u // n_c
            c = lax.rem(u, n_c)
            return [
                pltpu.make_async_copy(
                    tv_v.at[b, pl.ds(8 * r, 8)], out_hbm.at[t, r, c],
                    ssem.at[b])
                for r in range(D // 8)
            ]

        def transpose(b):
            @plsc.parallel_loop(0, 16, unroll=4)
            def _(s):
                base = (lane + s) & 15
                for k in range(CH // 16):
                    row = lane + 16 * k
                    for cb in range(D // 16):
                        col = 16 * cb + base
                        vec = plsc.load_gather(rows_v.at[b], [row, col])
                        plsc.store_scatter(tv_v.at[b], [col, row], vec)

        for b in range(NBUF):
            g_copy(b, b).start()

        @pl.loop(0, n_groups)
        def _(g):
            j0 = g * NBUF
            for b in range(NBUF):
                g_copy(j0 + b, b).wait()

                @pl.when(g > 0)
                def _():
                    for cp in s_copies(j0 + b - NBUF, b):
                        cp.wait()

                transpose(b)
                for cp in s_copies(j0 + b, b):
                    cp.start()

                @pl.when(g + 1 < n_groups)
                def _():
                    g_copy(j0 + NBUF + b, b).start()

        for b in range(NBUF):
            for cp in s_copies((n_groups - 1) * NBUF + b, b):
                cp.wait()

    return gather_kernel(W_embed, idx)


def kernel(x, W_embed):
    N, T = x.shape
    V = W_embed.shape[0]
    table = _sc_format(W_embed.T).reshape(V, D)
    idx = x.T.reshape(NW, (N // 128) * T // NW, 128).astype(jnp.int32)
    L = _sc_gather_t(table, idx)
    return L.transpose(2, 4, 0, 1, 3).reshape(N, T, D)


# speedup vs baseline: 1.0039x; 1.0039x over previous
import functools

import jax
import jax.numpy as jnp
from jax import lax
from jax.experimental import pallas as pl
from jax.experimental.pallas import tpu as pltpu
from jax.experimental.pallas import tpu_sc as plsc

NC = 2
NS = 16
NW = NC * NS
CH = 128
D = 32
NBUF = 5


PB = 6


@jax.jit
def _sc_format(Wt):
    V = Wt.shape[1]
    n_full = V // 128
    rem = V - n_full * 128
    n_groups = (n_full // NW + PB) // PB
    mesh = plsc.VectorSubcoreMesh(core_axis_name="c", subcore_axis_name="s")

    @functools.partial(
        pl.kernel,
        out_type=jax.ShapeDtypeStruct((V * D // 128, 128), jnp.float32),
        mesh=mesh,
        scratch_types=[
            pltpu.VMEM((PB, D, 128), jnp.float32),
            pltpu.VMEM((PB, D, 128), jnp.float32),
            pltpu.VMEM((D, rem), jnp.float32),
            pltpu.VMEM((rem * D // 128, 128), jnp.float32),
            pltpu.SemaphoreType.DMA((PB,)),
            pltpu.SemaphoreType.DMA((PB,)),
        ],
        compiler_params=pltpu.CompilerParams(
            use_tc_tiling_on_sc=True, needs_layout_passes=False),
    )
    def fmt_kernel(wt_hbm, out_hbm, w32, t128, w64, t64, rsem, wsem):
        wid = lax.axis_index("s") * NC + lax.axis_index("c")
        lane = jnp.arange(16, dtype=jnp.int32)

        def r_copy(u, b):
            return pltpu.make_async_copy(
                wt_hbm.at[:, pl.ds(u * 128, 128)], w32.at[b], rsem.at[b])

        def w_copy(u, b):
            return pltpu.make_async_copy(
                t128.at[b], out_hbm.at[pl.ds(u * D, D)], wsem.at[b])

        def transpose(src, dst, ni):
            @plsc.parallel_loop(0, 16, unroll=4)
            def _(s):
                shift = (lane + s) & 15
                lo = lane * D + shift
                for db in range(D // 16):
                    col_d = 16 * db + shift
                    hi0 = (lo + 16 * db) >> 7
                    lo0 = (lo + 16 * db) & 127
                    for ib in range(ni // 16):
                        row_i = lane + 16 * ib
                        vec = plsc.load_gather(src, [col_d, row_i])
                        plsc.store_scatter(
                            dst, [hi0 + (16 * D // 128) * ib, lo0], vec)

        for b in range(PB):
            @pl.when(wid + NW * b < n_full)
            def _():
                r_copy(wid + NW * b, b).start()

        @pl.loop(0, n_groups)
        def _(g):
            for b in range(PB):
                u = wid + NW * (g * PB + b)

                @pl.when(u < n_full)
                def _():
                    r_copy(u, b).wait()

                    @pl.when(u >= NW * PB)
                    def _():
                        w_copy(u - NW * PB, b).wait()

                    transpose(w32.at[b], t128.at[b], 128)
                    w_copy(u, b).start()

                    @pl.when(u + NW * PB < n_full)
                    def _():
                        r_copy(u + NW * PB, b).start()

        for b in range(PB):
            w_copy(0, b).wait()

        if rem:
            @pl.when(wid == lax.rem(n_full, NW))
            def _():
                pltpu.sync_copy(wt_hbm.at[:, pl.ds(n_full * 128, rem)], w64)
                transpose(w64, t64, rem)
                pltpu.sync_copy(
                    t64, out_hbm.at[pl.ds(n_full * D, rem * D // 128)])

    return fmt_kernel(Wt)


@jax.jit
def _sc_gather_t(W_embed, idx):
    n_t, n_c = 200, 32
    n_units = n_t * n_c
    du = n_units // NW
    n_groups = du // NBUF
    mesh = plsc.VectorSubcoreMesh(core_axis_name="c", subcore_axis_name="s")

    @functools.partial(
        pl.kernel,
        out_type=jax.ShapeDtypeStruct((n_t, D // 8, n_c, 8, CH), jnp.float32),
        mesh=mesh,
        scratch_types=[
            pltpu.VMEM((du, CH), jnp.int32),
            pltpu.VMEM((NBUF, CH, D), jnp.float32),
            pltpu.VMEM((NBUF, D, CH), jnp.float32),
            pltpu.SemaphoreType.DMA((NBUF,)),
            pltpu.SemaphoreType.DMA((NBUF,)),
        ],
        compiler_params=pltpu.CompilerParams(
            use_tc_tiling_on_sc=False, needs_layout_passes=False),
    )
    def gather_kernel(table_hbm, idx_hbm, out_hbm, idx_v, rows_v, tv_v, gsem, ssem):
        wid = lax.axis_index("s") * NC + lax.axis_index("c")
        u0 = wid * du
        pltpu.sync_copy(idx_hbm.at[wid], idx_v)

        lane = jnp.arange(16, dtype=jnp.int32)

        def g_copy(j, b):
            return pltpu.make_async_copy(
                table_hbm.at[idx_v.at[j]], rows_v.at[b], gsem.at[b])

        def s_copies(j, b):
            u = u0 + j
            t = u // n_c
            c = lax.rem(u, n_c)
            return [
                pltpu.make_async_copy(
                    tv_v.at[b, pl.ds(8 * r, 8)], out_hbm.at[t, r, c],
                    ssem.at[b])
                for r in range(D // 8)
            ]

        def transpose(b):
            @plsc.parallel_loop(0, 16, unroll=4)
            def _(s):
                base = (lane + s) & 15
                for k in range(CH // 16):
                    row = lane + 16 * k
                    for cb in range(D // 16):
                        col = 16 * cb + base
                        vec = plsc.load_gather(rows_v.at[b], [row, col])
                        plsc.store_scatter(tv_v.at[b], [col, row], vec)

        for b in range(NBUF):
            g_copy(b, b).start()

        @pl.loop(0, n_groups)
        def _(g):
            j0 = g * NBUF
            for b in range(NBUF):
                g_copy(j0 + b, b).wait()

                @pl.when(g > 0)
                def _():
                    for cp in s_copies(j0 + b - NBUF, b):
                        cp.wait()

                transpose(b)
                for cp in s_copies(j0 + b, b):
                    cp.start()

                @pl.when(g + 1 < n_groups)
                def _():
                    g_copy(j0 + NBUF + b, b).start()

        for b in range(NBUF):
            for cp in s_copies((n_groups - 1) * NBUF + b, b):
                cp.wait()

    return gather_kernel(W_embed, idx)


def kernel(x, W_embed):
    N, T = x.shape
    V = W_embed.shape[0]
    table = _sc_format(W_embed.T).reshape(V, D)
    idx = x.T.reshape(NW, (N // 128) * T // NW, 128).astype(jnp.int32)
    L = _sc_gather_t(table, idx)
    return L.transpose(2, 4, 0, 1, 3).reshape(N, T, D)


# speedup vs baseline: 1.1415x; 1.1370x over previous
import functools

import jax
import jax.numpy as jnp
from jax import lax
from jax.experimental import pallas as pl
from jax.experimental.pallas import tpu as pltpu
from jax.experimental.pallas import tpu_sc as plsc

NC = 2
NS = 16
NW = NC * NS
CH = 128
D = 32
NBUF = 4


PB = 4


@jax.jit
def _sc_format(Wt):
    V = Wt.shape[1]
    n_full = V // 128
    rem = V - n_full * 128
    n_groups = (n_full // NW + PB) // PB
    mesh = plsc.VectorSubcoreMesh(core_axis_name="c", subcore_axis_name="s")

    @functools.partial(
        pl.kernel,
        out_type=jax.ShapeDtypeStruct((V * D // 128, 128), jnp.float32),
        mesh=mesh,
        scratch_types=[
            pltpu.VMEM((PB, D, 128), jnp.float32),
            pltpu.VMEM((PB, D, 128), jnp.float32),
            pltpu.VMEM((D, rem), jnp.float32),
            pltpu.VMEM((rem * D // 128, 128), jnp.float32),
            pltpu.SemaphoreType.DMA((PB,)),
            pltpu.SemaphoreType.DMA((PB,)),
        ],
        compiler_params=pltpu.CompilerParams(
            use_tc_tiling_on_sc=True, needs_layout_passes=False),
    )
    def fmt_kernel(wt_hbm, out_hbm, w32, t128, w64, t64, rsem, wsem):
        wid = lax.axis_index("s") * NC + lax.axis_index("c")
        lane = jnp.arange(16, dtype=jnp.int32)

        def r_copy(u, b):
            return pltpu.make_async_copy(
                wt_hbm.at[:, pl.ds(u * 128, 128)], w32.at[b], rsem.at[b])

        def w_copy(u, b):
            return pltpu.make_async_copy(
                t128.at[b], out_hbm.at[pl.ds(u * D, D)], wsem.at[b])

        def transpose(src, dst, ni):
            @plsc.parallel_loop(0, 16, unroll=4)
            def _(s):
                shift = (lane + s) & 15
                lo = lane * D + shift
                for db in range(D // 16):
                    col_d = 16 * db + shift
                    hi0 = (lo + 16 * db) >> 7
                    lo0 = (lo + 16 * db) & 127
                    for ib in range(ni // 16):
                        row_i = lane + 16 * ib
                        vec = plsc.load_gather(src, [col_d, row_i])
                        plsc.store_scatter(
                            dst, [hi0 + (16 * D // 128) * ib, lo0], vec)

        for b in range(PB):
            @pl.when(wid + NW * b < n_full)
            def _():
                r_copy(wid + NW * b, b).start()

        @pl.loop(0, n_groups)
        def _(g):
            for b in range(PB):
                u = wid + NW * (g * PB + b)

                @pl.when(u < n_full)
                def _():
                    r_copy(u, b).wait()

                    @pl.when(u >= NW * PB)
                    def _():
                        w_copy(u - NW * PB, b).wait()

                    transpose(w32.at[b], t128.at[b], 128)
                    w_copy(u, b).start()

                    @pl.when(u + NW * PB < n_full)
                    def _():
                        r_copy(u + NW * PB, b).start()

        for b in range(PB):
            w_copy(0, b).wait()

        if rem:
            @pl.when(wid == lax.rem(n_full, NW))
            def _():
                pltpu.sync_copy(wt_hbm.at[:, pl.ds(n_full * 128, rem)], w64)
                transpose(w64, t64, rem)
                pltpu.sync_copy(
                    t64, out_hbm.at[pl.ds(n_full * D, rem * D // 128)])

    return fmt_kernel(Wt)


@jax.jit
def _sc_gather_t(W_embed, idx):
    n_t, n_c = 200, 32
    n_units = n_t * n_c
    du = n_units // NW
    n_groups = du // NBUF
    mesh = plsc.VectorSubcoreMesh(core_axis_name="c", subcore_axis_name="s")

    @functools.partial(
        pl.kernel,
        out_type=jax.ShapeDtypeStruct((n_t, D // 8, n_c, 8, CH), jnp.float32),
        mesh=mesh,
        scratch_types=[
            pltpu.VMEM((du, CH), jnp.int32),
            pltpu.VMEM((NBUF, CH, D), jnp.float32),
            pltpu.VMEM((NBUF, D, CH), jnp.float32),
            pltpu.SemaphoreType.DMA((NBUF,)),
            pltpu.SemaphoreType.DMA((NBUF,)),
        ],
        compiler_params=pltpu.CompilerParams(
            use_tc_tiling_on_sc=False, needs_layout_passes=False),
    )
    def gather_kernel(table_hbm, idx_hbm, out_hbm, idx_v, rows_v, tv_v, gsem, ssem):
        wid = lax.axis_index("s") * NC + lax.axis_index("c")
        u0 = wid * du
        pltpu.sync_copy(idx_hbm.at[wid], idx_v)

        lane = jnp.arange(16, dtype=jnp.int32)

        def g_copy(j, b):
            return pltpu.make_async_copy(
                table_hbm.at[idx_v.at[j]], rows_v.at[b], gsem.at[b])

        def s_copies(j, b):
            u = u0 + j
            t = u // n_c
            c = lax.rem(u, n_c)
            return [
                pltpu.make_async_copy(
                    tv_v.at[b, pl.ds(8 * r, 8)], out_hbm.at[t, r, c],
                    ssem.at[b])
                for r in range(D // 8)
            ]

        def transpose(b):
            @plsc.parallel_loop(0, 16, unroll=4)
            def _(s):
                base = (lane + s) & 15
                for k in range(CH // 16):
                    row = lane + 16 * k
                    for cb in range(D // 16):
                        col = 16 * cb + base
                        vec = plsc.load_gather(rows_v.at[b], [row, col])
                        plsc.store_scatter(tv_v.at[b], [col, row], vec)

        for b in range(NBUF):
            g_copy(b, b).start()

        @pl.loop(0, n_groups)
        def _(g):
            j0 = g * NBUF
            for b in range(NBUF):
                g_copy(j0 + b, b).wait()

                @pl.when(g > 0)
                def _():
                    for cp in s_copies(j0 + b - NBUF, b):
                        cp.wait()

                transpose(b)
                for cp in s_copies(j0 + b, b):
                    cp.start()

                @pl.when(g + 1 < n_groups)
                def _():
                    g_copy(j0 + NBUF + b, b).start()

        for b in range(NBUF):
            for cp in s_copies((n_groups - 1) * NBUF + b, b):
                cp.wait()

    return gather_kernel(W_embed, idx)


def kernel(x, W_embed):
    N, T = x.shape
    V = W_embed.shape[0]
    table = _sc_format(W_embed.T).reshape(V, D)
    idx = x.T.reshape(NW, (N // 128) * T // NW, 128).astype(jnp.int32)
    L = _sc_gather_t(table, idx)
    return L.transpose(2, 4, 0, 1, 3).reshape(N, T, D)
